# Initial kernel scaffold; baseline (speedup 1.0000x reference)
import jax
import jax.numpy as jnp
from jax.experimental import pallas as pl


def kernel(word_table, node_table, Wc, bc, Wih0, Whh0, bih0, bhh0, Wih1, Whh1, bih1, bhh1, Wm1, bm1, Wm2, bm2, word_ids, node_ids, parent_idx, segment_ids):
    raise NotImplementedError("write your pallas kernel here")


# speedup vs baseline: 2.1828x; 2.1828x over previous
import functools

import jax
import jax.numpy as jnp
from jax.experimental import pallas as pl
from jax.experimental.pallas import tpu as pltpu

B = 16
H = 128
G3 = 3 * H


def _gru_update(gi, gh, h):
    ir, iz, inn = gi[:, :H], gi[:, H:2 * H], gi[:, 2 * H:]
    hr, hz, hn = gh[:, :H], gh[:, H:2 * H], gh[:, 2 * H:]
    r = jax.nn.sigmoid(ir + hr)
    z = jax.nn.sigmoid(iz + hz)
    n = jnp.tanh(inn + r * hn)
    return (1.0 - z) * n + z * h


def _bigru_max_body(xs_ref, giw0_ref, bih0_ref, whh0_ref, bhh0_ref,
                    giw1_ref, bih1_ref, whh1_ref, bhh1_ref,
                    out_ref, l0_ref, gi_ref):
    T = xs_ref.shape[0]
    Din = xs_ref.shape[2]

    xs = xs_ref[...].reshape(T * B, Din)
    gi0 = jnp.dot(xs, giw0_ref[...], preferred_element_type=jnp.float32)
    gi_ref[...] = (gi0 + bih0_ref[...]).reshape(T, B, 2 * G3)

    def step0(t, carry):
        hf, hb = carry
        gh = jnp.dot(jnp.concatenate([hf, hb], axis=1), whh0_ref[...],
                     preferred_element_type=jnp.float32) + bhh0_ref[...]
        gif = gi_ref[t, :, :G3]
        gib = gi_ref[T - 1 - t, :, G3:]
        hf = _gru_update(gif, gh[:, :G3], hf)
        hb = _gru_update(gib, gh[:, G3:], hb)
        l0_ref[t, :, :H] = hf
        l0_ref[T - 1 - t, :, H:] = hb
        return hf, hb

    z16 = jnp.zeros((B, H), jnp.float32)
    jax.lax.fori_loop(0, T, step0, (z16, z16))

    l0 = l0_ref[...].reshape(T * B, 2 * H)
    gi1 = jnp.dot(l0, giw1_ref[...], preferred_element_type=jnp.float32)
    gi_ref[...] = (gi1 + bih1_ref[...]).reshape(T, B, 2 * G3)

    def step1(t, carry):
        hf, hb, acc = carry
        gh = jnp.dot(jnp.concatenate([hf, hb], axis=1), whh1_ref[...],
                     preferred_element_type=jnp.float32) + bhh1_ref[...]
        gif = gi_ref[t, :, :G3]
        gib = gi_ref[T - 1 - t, :, G3:]
        hf = _gru_update(gif, gh[:, :G3], hf)
        hb = _gru_update(gib, gh[:, G3:], hb)
        acc = jnp.maximum(acc, jnp.concatenate([hf, hb], axis=1))
        return hf, hb, acc

    ninf = jnp.full((B, 2 * H), -jnp.inf, jnp.float32)
    _, _, acc = jax.lax.fori_loop(0, T, step1, (z16, z16, ninf))
    out_ref[...] = acc


def _bigru_maxpool(xs_tm, giw0, bih0c, whh0b, bhh0c, giw1, bih1c, whh1b, bhh1c,
                   interpret=False):
    T = xs_tm.shape[0]
    return pl.pallas_call(
        _bigru_max_body,
        out_shape=jax.ShapeDtypeStruct((B, 2 * H), jnp.float32),
        scratch_shapes=[
            pltpu.VMEM((T, B, 2 * H), jnp.float32),
            pltpu.VMEM((T, B, 2 * G3), jnp.float32),
        ],
        interpret=interpret,
    )(xs_tm, giw0, bih0c, whh0b, bhh0c, giw1, bih1c, whh1b, bhh1c)


def _pack_dir_weights(Wih, bih, Whh, bhh):
    giw = jnp.concatenate([Wih[0], Wih[1]], axis=1)
    bihc = jnp.concatenate([bih[0], bih[1]])
    Din = Whh.shape[1]
    whhb = jnp.zeros((2 * Din, 2 * G3), jnp.float32)
    whhb = whhb.at[:Din, :G3].set(Whh[0]).at[Din:, G3:].set(Whh[1])
    bhhc = jnp.concatenate([bhh[0], bhh[1]])
    return giw, bihc, whhb, bhhc


def kernel(word_table, node_table, Wc, bc, Wih0, Whh0, bih0, bhh0,
           Wih1, Whh1, bih1, bhh1, Wm1, bm1, Wm2, bm2,
           word_ids, node_ids, parent_idx, segment_ids):
    S = 512
    giw0, bih0c, whh0b, bhh0c = _pack_dir_weights(Wih0, bih0, Whh0, bhh0)
    giw1, bih1c, whh1b, bhh1c = _pack_dir_weights(Wih1, bih1, Whh1, bhh1)

    word_emb = word_table[word_ids]
    word_tm = jnp.swapaxes(word_emb, 0, 1)
    word_vec = _bigru_maxpool(word_tm, giw0, bih0c, whh0b, bhh0c,
                              giw1, bih1c, whh1b, bhh1c)

    node_emb = node_table[node_ids]
    node_h = node_emb @ Wc + bc
    agg = jnp.zeros_like(node_h).at[parent_idx].add(node_h)
    node_h = node_h + agg
    stmt = jax.ops.segment_max(node_h, segment_ids, num_segments=B * S)
    stmt = jnp.where(jnp.isfinite(stmt), stmt, 0.0)
    stmt_tm = jnp.swapaxes(stmt.reshape(B, S, -1), 0, 1)
    code_vec = _bigru_maxpool(stmt_tm, giw0, bih0c, whh0b, bhh0c,
                              giw1, bih1c, whh1b, bhh1c)

    dot = jnp.sum(word_vec * code_vec, axis=-1, keepdims=True)
    nw = jnp.sqrt(jnp.sum(word_vec ** 2, axis=-1, keepdims=True))
    nc = jnp.sqrt(jnp.sum(code_vec ** 2, axis=-1, keepdims=True))
    cos = dot / (nw * nc + 1e-8)
    diff = word_vec - code_vec
    l2 = jnp.sqrt(jnp.sum(diff ** 2, axis=-1, keepdims=True) + 1e-12)
    l1 = jnp.sum(jnp.abs(diff), axis=-1, keepdims=True)
    feats = jnp.concatenate([cos, dot, l2, l1], axis=-1)
    hid = jax.nn.relu(feats @ Wm1 + bm1)
    score = hid @ Wm2 + bm2
    return score


# speedup vs baseline: 2.9587x; 1.3555x over previous
import functools

import jax
import jax.numpy as jnp
from jax import lax
from jax.experimental import pallas as pl
from jax.experimental.pallas import tpu as pltpu
from jax.experimental.pallas import tpu_sc as plsc

B = 16
H = 128
G3 = 3 * H
NC = 2
NS = 16
NW = NC * NS


def _proj_body(tab_ref, wc_ref, bc_ref, out_ref):
    out_ref[...] = jnp.dot(tab_ref[...], wc_ref[...],
                           preferred_element_type=jnp.float32) + bc_ref[...]


def _project_table(table, Wc, bc):
    V, D = table.shape
    E = Wc.shape[1]
    blk = 2000
    assert V % blk == 0
    return pl.pallas_call(
        _proj_body,
        grid=(V // blk,),
        in_specs=[
            pl.BlockSpec((blk, D), lambda i: (i, 0)),
            pl.BlockSpec((D, E), lambda i: (0, 0)),
            pl.BlockSpec((1, E), lambda i: (0, 0)),
        ],
        out_specs=pl.BlockSpec((blk, E), lambda i: (i, 0)),
        out_shape=jax.ShapeDtypeStruct((V, E), jnp.float32),
    )(table, Wc, bc.reshape(1, E))


def _sc_gather(table, ids):
    R = ids.shape[0]
    D = table.shape[1]
    assert R % (NW * 128) == 0
    K = R // (NW * 128)
    idx2d = ids.reshape(R // 128, 128)
    mesh = plsc.VectorSubcoreMesh(core_axis_name="c", subcore_axis_name="s")

    def body(table_hbm, idx_hbm, out_hbm, idx_v, rows_v, sem):
        wid = lax.axis_index("s") * NC + lax.axis_index("c")
        pltpu.sync_copy(idx_hbm.at[pl.ds(wid * K, K)], idx_v)

        def chunk(k, carry):
            pltpu.async_copy(table_hbm.at[idx_v.at[k]], rows_v, sem).wait()
            pltpu.sync_copy(rows_v,
                            out_hbm.at[pl.ds((wid * K + k) * 128, 128)])
            return carry

        lax.fori_loop(0, K, chunk, 0)

    f = pl.kernel(
        body,
        out_type=jax.ShapeDtypeStruct((R, D), jnp.float32),
        mesh=mesh,
        scratch_types=[
            pltpu.VMEM((K, 128), jnp.int32),
            pltpu.VMEM((128, D), jnp.float32),
            pltpu.SemaphoreType.DMA,
        ],
    )
    return f(table, idx2d)


def _gru_update(gi, gh, h):
    ir, iz, inn = gi[:, :H], gi[:, H:2 * H], gi[:, 2 * H:]
    hr, hz, hn = gh[:, :H], gh[:, H:2 * H], gh[:, 2 * H:]
    r = jax.nn.sigmoid(ir + hr)
    z = jax.nn.sigmoid(iz + hz)
    n = jnp.tanh(inn + r * hn)
    return (1.0 - z) * n + z * h


def _bigru_max_body(xs_ref, giw0_ref, bih0_ref, whh0_ref, bhh0_ref,
                    giw1_ref, bih1_ref, whh1_ref, bhh1_ref,
                    out_ref, l0_ref, gi_ref):
    T = xs_ref.shape[0]
    Din = xs_ref.shape[2]

    xs = xs_ref[...].reshape(T * B, Din)
    gi0 = jnp.dot(xs, giw0_ref[...], preferred_element_type=jnp.float32)
    gi_ref[...] = (gi0 + bih0_ref[...]).reshape(T, B, 2 * G3)

    def step0(t, carry):
        hf, hb = carry
        gh = jnp.dot(jnp.concatenate([hf, hb], axis=1), whh0_ref[...],
                     preferred_element_type=jnp.float32) + bhh0_ref[...]
        gif = gi_ref[t, :, :G3]
        gib = gi_ref[T - 1 - t, :, G3:]
        hf = _gru_update(gif, gh[:, :G3], hf)
        hb = _gru_update(gib, gh[:, G3:], hb)
        l0_ref[t, :, :H] = hf
        l0_ref[T - 1 - t, :, H:] = hb
        return hf, hb

    z16 = jnp.zeros((B, H), jnp.float32)
    jax.lax.fori_loop(0, T, step0, (z16, z16))

    l0 = l0_ref[...].reshape(T * B, 2 * H)
    gi1 = jnp.dot(l0, giw1_ref[...], preferred_element_type=jnp.float32)
    gi_ref[...] = (gi1 + bih1_ref[...]).reshape(T, B, 2 * G3)

    def step1(t, carry):
        hf, hb, acc = carry
        gh = jnp.dot(jnp.concatenate([hf, hb], axis=1), whh1_ref[...],
                     preferred_element_type=jnp.float32) + bhh1_ref[...]
        gif = gi_ref[t, :, :G3]
        gib = gi_ref[T - 1 - t, :, G3:]
        hf = _gru_update(gif, gh[:, :G3], hf)
        hb = _gru_update(gib, gh[:, G3:], hb)
        acc = jnp.maximum(acc, jnp.concatenate([hf, hb], axis=1))
        return hf, hb, acc

    ninf = jnp.full((B, 2 * H), -jnp.inf, jnp.float32)
    _, _, acc = jax.lax.fori_loop(0, T, step1, (z16, z16, ninf))
    out_ref[...] = acc


def _bigru_maxpool(xs_tm, giw0, bih0c, whh0b, bhh0c, giw1, bih1c, whh1b, bhh1c,
                   interpret=False):
    T = xs_tm.shape[0]
    return pl.pallas_call(
        _bigru_max_body,
        out_shape=jax.ShapeDtypeStruct((B, 2 * H), jnp.float32),
        scratch_shapes=[
            pltpu.VMEM((T, B, 2 * H), jnp.float32),
            pltpu.VMEM((T, B, 2 * G3), jnp.float32),
        ],
        interpret=interpret,
    )(xs_tm, giw0, bih0c, whh0b, bhh0c, giw1, bih1c, whh1b, bhh1c)


def _pack_dir_weights(Wih, bih, Whh, bhh):
    giw = jnp.concatenate([Wih[0], Wih[1]], axis=1)
    bihc = jnp.concatenate([bih[0], bih[1]])
    Din = Whh.shape[1]
    whhb = jnp.zeros((2 * Din, 2 * G3), jnp.float32)
    whhb = whhb.at[:Din, :G3].set(Whh[0]).at[Din:, G3:].set(Whh[1])
    bhhc = jnp.concatenate([bhh[0], bhh[1]])
    return giw, bihc, whhb, bhhc


def kernel(word_table, node_table, Wc, bc, Wih0, Whh0, bih0, bhh0,
           Wih1, Whh1, bih1, bhh1, Wm1, bm1, Wm2, bm2,
           word_ids, node_ids, parent_idx, segment_ids):
    S = 512
    giw0, bih0c, whh0b, bhh0c = _pack_dir_weights(Wih0, bih0, Whh0, bhh0)
    giw1, bih1c, whh1b, bhh1c = _pack_dir_weights(Wih1, bih1, Whh1, bhh1)

    L = word_ids.shape[1]
    wids = jnp.concatenate(
        [word_ids.reshape(-1), jnp.zeros((4096 - word_ids.size,), jnp.int32)])
    word_emb = _sc_gather(word_table, wids)[:word_ids.size]
    word_tm = jnp.swapaxes(word_emb.reshape(B, L, -1), 0, 1)
    word_vec = _bigru_maxpool(word_tm, giw0, bih0c, whh0b, bhh0c,
                              giw1, bih1c, whh1b, bhh1c)

    proj = _project_table(node_table, Wc, bc)
    node_h = _sc_gather(proj, node_ids)
    agg = jnp.zeros_like(node_h).at[parent_idx].add(node_h)
    node_h = node_h + agg
    stmt = jax.ops.segment_max(node_h, segment_ids, num_segments=B * S)
    stmt = jnp.where(jnp.isfinite(stmt), stmt, 0.0)
    stmt_tm = jnp.swapaxes(stmt.reshape(B, S, -1), 0, 1)
    code_vec = _bigru_maxpool(stmt_tm, giw0, bih0c, whh0b, bhh0c,
                              giw1, bih1c, whh1b, bhh1c)

    dot = jnp.sum(word_vec * code_vec, axis=-1, keepdims=True)
    nw = jnp.sqrt(jnp.sum(word_vec ** 2, axis=-1, keepdims=True))
    nc = jnp.sqrt(jnp.sum(code_vec ** 2, axis=-1, keepdims=True))
    cos = dot / (nw * nc + 1e-8)
    diff = word_vec - code_vec
    l2 = jnp.sqrt(jnp.sum(diff ** 2, axis=-1, keepdims=True) + 1e-12)
    l1 = jnp.sum(jnp.abs(diff), axis=-1, keepdims=True)
    feats = jnp.concatenate([cos, dot, l2, l1], axis=-1)
    hid = jax.nn.relu(feats @ Wm1 + bm1)
    score = hid @ Wm2 + bm2
    return score


# speedup vs baseline: 3.0209x; 1.0210x over previous
import functools

import jax
import jax.numpy as jnp
from jax import lax
from jax.experimental import pallas as pl
from jax.experimental.pallas import tpu as pltpu
from jax.experimental.pallas import tpu_sc as plsc

B = 16
H = 128
G3 = 3 * H
NC = 2
NS = 16
NW = NC * NS


def _proj_body(tab_ref, wc_ref, bc_ref, out_ref):
    out_ref[...] = jnp.dot(tab_ref[...], wc_ref[...],
                           preferred_element_type=jnp.float32) + bc_ref[...]


def _project_table(table, Wc, bc):
    V, D = table.shape
    E = Wc.shape[1]
    blk = 2000
    assert V % blk == 0
    return pl.pallas_call(
        _proj_body,
        grid=(V // blk,),
        in_specs=[
            pl.BlockSpec((blk, D), lambda i: (i, 0)),
            pl.BlockSpec((D, E), lambda i: (0, 0)),
            pl.BlockSpec((1, E), lambda i: (0, 0)),
        ],
        out_specs=pl.BlockSpec((blk, E), lambda i: (i, 0)),
        out_shape=jax.ShapeDtypeStruct((V, E), jnp.float32),
    )(table, Wc, bc.reshape(1, E))


def _sc_gather(table, ids):
    R = ids.shape[0]
    D = table.shape[1]
    assert R % (NW * 128) == 0
    K = R // (NW * 128)
    idx2d = ids.reshape(R // 128, 128)
    mesh = plsc.VectorSubcoreMesh(core_axis_name="c", subcore_axis_name="s")

    def body(table_hbm, idx_hbm, out_hbm, idx_v, rows_v, sem):
        wid = lax.axis_index("s") * NC + lax.axis_index("c")
        pltpu.sync_copy(idx_hbm.at[pl.ds(wid * K, K)], idx_v)

        def chunk(k, carry):
            pltpu.async_copy(table_hbm.at[idx_v.at[k]], rows_v, sem).wait()
            pltpu.sync_copy(rows_v,
                            out_hbm.at[pl.ds((wid * K + k) * 128, 128)])
            return carry

        lax.fori_loop(0, K, chunk, 0)

    f = pl.kernel(
        body,
        out_type=jax.ShapeDtypeStruct((R, D), jnp.float32),
        mesh=mesh,
        scratch_types=[
            pltpu.VMEM((K, 128), jnp.int32),
            pltpu.VMEM((128, D), jnp.float32),
            pltpu.SemaphoreType.DMA,
        ],
    )
    return f(table, idx2d)


def _gru_update(gi, gh, h):
    ir, iz, inn = gi[:, :H], gi[:, H:2 * H], gi[:, 2 * H:]
    hr, hz, hn = gh[:, :H], gh[:, H:2 * H], gh[:, 2 * H:]
    r = jax.nn.sigmoid(ir + hr)
    z = jax.nn.sigmoid(iz + hz)
    n = jnp.tanh(inn + r * hn)
    return (1.0 - z) * n + z * h


def _bigru_max_body(xs_ref, giw0_ref, bih0_ref, whh0_ref, bhh0_ref,
                    giw1_ref, bih1_ref, whh1_ref, bhh1_ref,
                    out_ref, l0_ref, gi_ref):
    T = xs_ref.shape[0]
    Din = xs_ref.shape[2]

    xs = xs_ref[...].reshape(T * B, Din)
    gi0 = jnp.dot(xs, giw0_ref[...], preferred_element_type=jnp.float32)
    gi_ref[...] = (gi0 + bih0_ref[...]).reshape(T, B, 2 * G3)

    def step0(t, carry):
        hf, hb = carry
        gh = jnp.dot(jnp.concatenate([hf, hb], axis=1), whh0_ref[...],
                     preferred_element_type=jnp.float32) + bhh0_ref[...]
        gif = gi_ref[t, :, :G3]
        gib = gi_ref[T - 1 - t, :, G3:]
        hf = _gru_update(gif, gh[:, :G3], hf)
        hb = _gru_update(gib, gh[:, G3:], hb)
        l0_ref[t, :, :H] = hf
        l0_ref[T - 1 - t, :, H:] = hb
        return hf, hb

    z16 = jnp.zeros((B, H), jnp.float32)
    jax.lax.fori_loop(0, T, step0, (z16, z16))

    l0 = l0_ref[...].reshape(T * B, 2 * H)
    gi1 = jnp.dot(l0, giw1_ref[...], preferred_element_type=jnp.float32)
    gi_ref[...] = (gi1 + bih1_ref[...]).reshape(T, B, 2 * G3)

    def step1(t, carry):
        hf, hb, acc = carry
        gh = jnp.dot(jnp.concatenate([hf, hb], axis=1), whh1_ref[...],
                     preferred_element_type=jnp.float32) + bhh1_ref[...]
        gif = gi_ref[t, :, :G3]
        gib = gi_ref[T - 1 - t, :, G3:]
        hf = _gru_update(gif, gh[:, :G3], hf)
        hb = _gru_update(gib, gh[:, G3:], hb)
        acc = jnp.maximum(acc, jnp.concatenate([hf, hb], axis=1))
        return hf, hb, acc

    ninf = jnp.full((B, 2 * H), -jnp.inf, jnp.float32)
    _, _, acc = jax.lax.fori_loop(0, T, step1, (z16, z16, ninf))
    out_ref[...] = acc


def _bigru_maxpool(xs_tm, giw0, bih0c, whh0b, bhh0c, giw1, bih1c, whh1b, bhh1c,
                   interpret=False):
    T = xs_tm.shape[0]
    return pl.pallas_call(
        _bigru_max_body,
        out_shape=jax.ShapeDtypeStruct((B, 2 * H), jnp.float32),
        scratch_shapes=[
            pltpu.VMEM((T, B, 2 * H), jnp.float32),
            pltpu.VMEM((T, B, 2 * G3), jnp.float32),
        ],
        interpret=interpret,
    )(xs_tm, giw0, bih0c, whh0b, bhh0c, giw1, bih1c, whh1b, bhh1c)


N_NODES = 163840
NSEG = 8192
SCAN = N_NODES // NS
SEG_PER_W = NSEG // NW
BS = 256


def _segmax_body(nodeh_hbm, agg_hbm, seg_hbm, out_hbm,
                 sbuf, pc_local, pcall, cflat, segw, vbuf, idxw, outbuf,
                 spm_pc):
    cidx = lax.axis_index("c")
    sidx = lax.axis_index("s")
    w = sidx * NC + cidx
    iota = lax.iota(jnp.int32, 16)
    zvec = jnp.zeros((16,), jnp.float32)
    ninf = jnp.full((16,), -jnp.inf, jnp.float32)

    def zloop2(i, c):
        for j in range(8):
            outbuf[i, pl.ds(j * 16, 16)] = zvec
        return c
    lax.fori_loop(0, SEG_PER_W, zloop2, 0)

    base_row = sidx * SCAN
    pltpu.sync_copy(seg_hbm.at[pl.ds(base_row, SCAN + 16)], sbuf)

    def search(k, c):
        t = k * (NSEG // NW)
        cnt = jnp.int32(0)
        step = 8192
        while step >= 1:
            idx = cnt + step - 1
            v = sbuf[pl.ds(jnp.minimum(idx, SCAN - 1), 16)][0]
            take = (idx < SCAN) & (v < t)
            cnt = jnp.where(take, cnt + step, cnt)
            step //= 2
        pc_local[pl.ds(k, 16)] = jnp.full((16,), cnt, jnp.int32)
        return c
    lax.fori_loop(0, NW + 1, search, 0)

    pltpu.sync_copy(pc_local, spm_pc.at[sidx])
    plsc.subcore_barrier()
    pltpu.sync_copy(spm_pc, pcall)
    c0 = jnp.zeros((16,), jnp.int32)
    c1 = jnp.zeros((16,), jnp.int32)
    c2 = jnp.zeros((16,), jnp.int32)
    for t2 in range(NS):
        c0 = c0 + pcall[t2, pl.ds(0, 16)]
        c1 = c1 + pcall[t2, pl.ds(16, 16)]
        c2 = c2 + pcall[t2, pl.ds(32, 16)]
    cflat[pl.ds(0, 16)] = c0
    cflat[pl.ds(16, 16)] = c1
    cflat[pl.ds(32, 16)] = c2

    s0 = w * SEG_PER_W
    rbeg = cflat[pl.ds(w, 16)][0]
    rend = cflat[pl.ds(w + 1, 16)][0]

    def block(bi, carry):
        base = bi * BS
        pltpu.sync_copy(seg_hbm.at[pl.ds(base, BS + 16)], segw)
        pltpu.sync_copy(nodeh_hbm.at[pl.ds(base, BS)], vbuf)
        for h in range(2):
            for k in range(8):
                idxw[h, pl.ds(k * 16, 16)] = (base + h * 128 + k * 16) + iota
        for h in range(2):
            pltpu.sync_copy(agg_hbm.at[idxw.at[h]],
                            vbuf.at[pl.ds(h * 128, 128)], add=True)

        def row(r, c2):
            seg_prev = c2[0]
            acc = c2[1:]
            i = r - base
            seg = segw[pl.ds(i, 16)][0]
            off = seg - s0
            same = seg == seg_prev
            new = []
            for j in range(8):
                v = vbuf[i, pl.ds(j * 16, 16)]
                aj = jnp.maximum(jnp.where(same, acc[j], ninf), v)
                outbuf[off, pl.ds(j * 16, 16)] = aj
                new.append(aj)
            return (seg,) + tuple(new)

        lo = jnp.maximum(rbeg, base)
        hi = jnp.minimum(rend, base + BS)
        return lax.fori_loop(lo, hi, row, carry)

    carry0 = (jnp.int32(-1),) + tuple(ninf for _ in range(8))
    lax.fori_loop(rbeg // BS, (rend + BS - 1) // BS, block, carry0)
    pltpu.sync_copy(outbuf, out_hbm.at[pl.ds(s0, SEG_PER_W)])


def _sc_segmax(nodeh, agg, seg_ext):
    mesh = plsc.VectorSubcoreMesh(core_axis_name="c", subcore_axis_name="s")
    f = pl.kernel(
        _segmax_body,
        out_type=jax.ShapeDtypeStruct((NSEG, 128), jnp.float32),
        mesh=mesh,
        scratch_types=[
            pltpu.VMEM((SCAN + 16,), jnp.int32),
            pltpu.VMEM((128,), jnp.int32),
            pltpu.VMEM((NS, 128), jnp.int32),
            pltpu.VMEM((48,), jnp.int32),
            pltpu.VMEM((BS + 16,), jnp.int32),
            pltpu.VMEM((BS, 128), jnp.float32),
            pltpu.VMEM((2, 128), jnp.int32),
            pltpu.VMEM((SEG_PER_W, 128), jnp.float32),
            pltpu.VMEM_SHARED((NS, 128), jnp.int32),
        ],
    )
    return f(nodeh, agg, seg_ext)


def _pack_dir_weights(Wih, bih, Whh, bhh):
    giw = jnp.concatenate([Wih[0], Wih[1]], axis=1)
    bihc = jnp.concatenate([bih[0], bih[1]])
    Din = Whh.shape[1]
    whhb = jnp.zeros((2 * Din, 2 * G3), jnp.float32)
    whhb = whhb.at[:Din, :G3].set(Whh[0]).at[Din:, G3:].set(Whh[1])
    bhhc = jnp.concatenate([bhh[0], bhh[1]])
    return giw, bihc, whhb, bhhc


def kernel(word_table, node_table, Wc, bc, Wih0, Whh0, bih0, bhh0,
           Wih1, Whh1, bih1, bhh1, Wm1, bm1, Wm2, bm2,
           word_ids, node_ids, parent_idx, segment_ids):
    S = 512
    giw0, bih0c, whh0b, bhh0c = _pack_dir_weights(Wih0, bih0, Whh0, bhh0)
    giw1, bih1c, whh1b, bhh1c = _pack_dir_weights(Wih1, bih1, Whh1, bhh1)

    L = word_ids.shape[1]
    wids = jnp.concatenate(
        [word_ids.reshape(-1), jnp.zeros((4096 - word_ids.size,), jnp.int32)])
    word_emb = _sc_gather(word_table, wids)[:word_ids.size]
    word_tm = jnp.swapaxes(word_emb.reshape(B, L, -1), 0, 1)
    word_vec = _bigru_maxpool(word_tm, giw0, bih0c, whh0b, bhh0c,
                              giw1, bih1c, whh1b, bhh1c)

    proj = _project_table(node_table, Wc, bc)
    node_h = _sc_gather(proj, node_ids)
    agg = jnp.zeros_like(node_h).at[parent_idx].add(node_h)
    seg_ext = jnp.concatenate(
        [segment_ids, jnp.full((BS,), NSEG, jnp.int32)])
    stmt = _sc_segmax(node_h, agg, seg_ext)
    stmt_tm = jnp.swapaxes(stmt.reshape(B, S, -1), 0, 1)
    code_vec = _bigru_maxpool(stmt_tm, giw0, bih0c, whh0b, bhh0c,
                              giw1, bih1c, whh1b, bhh1c)

    dot = jnp.sum(word_vec * code_vec, axis=-1, keepdims=True)
    nw = jnp.sqrt(jnp.sum(word_vec ** 2, axis=-1, keepdims=True))
    nc = jnp.sqrt(jnp.sum(code_vec ** 2, axis=-1, keepdims=True))
    cos = dot / (nw * nc + 1e-8)
    diff = word_vec - code_vec
    l2 = jnp.sqrt(jnp.sum(diff ** 2, axis=-1, keepdims=True) + 1e-12)
    l1 = jnp.sum(jnp.abs(diff), axis=-1, keepdims=True)
    feats = jnp.concatenate([cos, dot, l2, l1], axis=-1)
    hid = jax.nn.relu(feats @ Wm1 + bm1)
    score = hid @ Wm2 + bm2
    return score
